# baseline (speedup 1.0000x reference)
import functools

import jax
import jax.numpy as jnp
from jax.experimental import pallas as pl

HIDDEN = 128
NF = 128
NG = 50
NI = 6
CUTOFF = 5.0
MAXNB = 32
N_ATOMS = 4096

_LOG2 = 0.6931471805599453
_STEP = CUTOFF / (NG - 1)
_GAMMA = 0.5 / _STEP ** 2

_T = 256
_ET = _T * MAXNB

_INTERPRET = False


def _ssp(x):
    return jax.nn.softplus(x) - _LOG2


def _offsets_lanes(shape):
    lane_i = jax.lax.broadcasted_iota(jnp.int32, shape, len(shape) - 1)
    lane = lane_i.astype(jnp.float32)
    off = lane * _STEP
    return jnp.where(lane_i < NG, off, 1e9)


def _edge_fwd_body(ew_ref, cm_ref, g_ref, w0t_ref, b0_ref, w2t_ref, b2_ref,
                   agg_ref):
    ew = ew_ref[...]
    ew3 = jax.lax.broadcast_in_dim(ew, (_T, MAXNB, 128), (0, 1))
    off3 = _offsets_lanes((_T, MAXNB, 128))
    attr = jnp.exp(-_GAMMA * (ew3 - off3) ** 2).reshape(_ET, 128)
    Z = jnp.dot(attr, w0t_ref[...], preferred_element_type=jnp.float32) + b0_ref[...]
    A = _ssp(Z)
    W = jnp.dot(A, w2t_ref[...], preferred_element_type=jnp.float32) + b2_ref[...]
    cm3 = jax.lax.broadcast_in_dim(cm_ref[...], (_T, MAXNB, 128), (0, 1))
    msg = (W * g_ref[...]).reshape(_T, MAXNB, 128) * cm3
    agg_ref[...] = msg.sum(axis=1)


def _edge_bwd_body(ew_ref, cm_ref, mk_ref, g_ref, du_ref,
                   w0t_ref, b0_ref, w2t_ref, b2_ref, w0_ref, w2_ref,
                   dg_ref, dew_ref):
    ew = ew_ref[...]
    ew3 = jax.lax.broadcast_in_dim(ew, (_T, MAXNB, 128), (0, 1))
    off3 = _offsets_lanes((_T, MAXNB, 128))
    attr3 = jnp.exp(-_GAMMA * (ew3 - off3) ** 2)
    attr = attr3.reshape(_ET, 128)
    Z = jnp.dot(attr, w0t_ref[...], preferred_element_type=jnp.float32) + b0_ref[...]
    sigZ = jax.nn.sigmoid(Z)
    A = _ssp(Z)
    W = jnp.dot(A, w2t_ref[...], preferred_element_type=jnp.float32) + b2_ref[...]

    cm3 = jax.lax.broadcast_in_dim(cm_ref[...], (_T, MAXNB, 128), (0, 1))
    cm_e = cm3.reshape(_ET, 128)
    du3 = jax.lax.broadcast_in_dim(du_ref[...], (_T, MAXNB, 128), (0, 2))
    du_e = du3.reshape(_ET, 128)
    g = g_ref[...]

    cdu = cm_e * du_e
    dW = cdu * g
    dg_ref[...] = cdu * W
    dcm = (du_e * W * g).reshape(_T, MAXNB, 128).sum(axis=2)

    dA = jnp.dot(dW, w2_ref[...], preferred_element_type=jnp.float32)
    dZ = dA * sigZ
    dattr = jnp.dot(dZ, w0_ref[...], preferred_element_type=jnp.float32)
    dew_attr = (dattr.reshape(_T, MAXNB, 128) * attr3
                * (-2.0 * _GAMMA) * (ew3 - off3)).sum(axis=2)
    dc = mk_ref[...] * dcm
    dew_ref[...] = dew_attr + dc * (-0.5 * jnp.pi / CUTOFF) * jnp.sin(
        ew * (jnp.pi / CUTOFF))


def _pad_rows(w, rows):
    return jnp.zeros((rows, w.shape[1]), w.dtype).at[:w.shape[0]].set(w)


def _pad_cols(w, cols):
    return jnp.zeros((w.shape[0], cols), w.dtype).at[:, :w.shape[1]].set(w)


@functools.partial(jax.jit, static_argnames=())
def _edge_fwd(ew, cm, g, w0t, b0, w2t, b2):
    N = ew.shape[0]
    grid = N // _T
    return pl.pallas_call(
        _edge_fwd_body,
        grid=(grid,),
        in_specs=[
            pl.BlockSpec((_T, MAXNB), lambda i: (i, 0)),
            pl.BlockSpec((_T, MAXNB), lambda i: (i, 0)),
            pl.BlockSpec((_ET, 128), lambda i: (i, 0)),
            pl.BlockSpec((128, 128), lambda i: (0, 0)),
            pl.BlockSpec((1, 128), lambda i: (0, 0)),
            pl.BlockSpec((128, 128), lambda i: (0, 0)),
            pl.BlockSpec((1, 128), lambda i: (0, 0)),
        ],
        out_specs=pl.BlockSpec((_T, 128), lambda i: (i, 0)),
        out_shape=jax.ShapeDtypeStruct((N, 128), jnp.float32),
        interpret=_INTERPRET,
    )(ew, cm, g, w0t, b0, w2t, b2)


@functools.partial(jax.jit, static_argnames=())
def _edge_bwd(ew, cm, mk, g, du, w0t, b0, w2t, b2, w0, w2):
    N = ew.shape[0]
    grid = N // _T
    return pl.pallas_call(
        _edge_bwd_body,
        grid=(grid,),
        in_specs=[
            pl.BlockSpec((_T, MAXNB), lambda i: (i, 0)),
            pl.BlockSpec((_T, MAXNB), lambda i: (i, 0)),
            pl.BlockSpec((_T, MAXNB), lambda i: (i, 0)),
            pl.BlockSpec((_ET, 128), lambda i: (i, 0)),
            pl.BlockSpec((_T, 128), lambda i: (i, 0)),
            pl.BlockSpec((128, 128), lambda i: (0, 0)),
            pl.BlockSpec((1, 128), lambda i: (0, 0)),
            pl.BlockSpec((128, 128), lambda i: (0, 0)),
            pl.BlockSpec((1, 128), lambda i: (0, 0)),
            pl.BlockSpec((128, 128), lambda i: (0, 0)),
            pl.BlockSpec((128, 128), lambda i: (0, 0)),
        ],
        out_specs=[
            pl.BlockSpec((_ET, 128), lambda i: (i, 0)),
            pl.BlockSpec((_T, MAXNB), lambda i: (i, 0)),
        ],
        out_shape=[
            jax.ShapeDtypeStruct((N * MAXNB, 128), jnp.float32),
            jax.ShapeDtypeStruct((N, MAXNB), jnp.float32),
        ],
        interpret=_INTERPRET,
    )(ew, cm, mk, g, du, w0t, b0, w2t, b2, w0, w2)


_TR = 256


def _topk_body(dist_ref, dsel_ref, order_ref):
    row = dist_ref[...]
    n = row.shape[1]
    lane = jax.lax.broadcasted_iota(jnp.int32, (_TR, n), 1)
    big_i = jnp.int32(2 ** 30)
    dsel_cols = []
    order_cols = []
    for k in range(MAXNB):
        m = jnp.min(row, axis=1, keepdims=True)
        hit = row == m
        idx = jnp.min(jnp.where(hit, lane, big_i), axis=1, keepdims=True)
        dsel_cols.append(m)
        order_cols.append(idx)
        row = jnp.where(lane == idx, jnp.inf, row)
    dsel_ref[...] = jnp.concatenate(dsel_cols, axis=1)
    order_ref[...] = jnp.concatenate(order_cols, axis=1)


@functools.partial(jax.jit, static_argnames=())
def _topk(dist):
    N = dist.shape[0]
    return pl.pallas_call(
        _topk_body,
        grid=(N // _TR,),
        in_specs=[pl.BlockSpec((_TR, N), lambda i: (i, 0))],
        out_specs=[
            pl.BlockSpec((_TR, MAXNB), lambda i: (i, 0)),
            pl.BlockSpec((_TR, MAXNB), lambda i: (i, 0)),
        ],
        out_shape=[
            jax.ShapeDtypeStruct((N, MAXNB), jnp.float32),
            jax.ShapeDtypeStruct((N, MAXNB), jnp.int32),
        ],
        interpret=_INTERPRET,
    )(dist)


def _graph(pos):
    N = pos.shape[0]
    sq = jnp.sum(pos ** 2, axis=-1)
    d2 = sq[:, None] + sq[None, :] - 2.0 * pos @ pos.T
    d2 = jnp.maximum(d2, 0.0)
    dist = jnp.sqrt(d2)
    dist = jnp.where(jnp.eye(N, dtype=bool), jnp.inf, dist)
    d_sel, order = _topk(dist)
    mask = d_sel < CUTOFF
    centers = jnp.arange(N, dtype=order.dtype)[:, None]
    nbr = jnp.where(mask, order, centers).astype(jnp.int32)
    return nbr, mask


def kernel(z, pos, params):
    N = pos.shape[0]
    nbr, mask = _graph(pos)
    nbr_flat = nbr.reshape(-1)
    maskf = mask.astype(jnp.float32)

    delta = pos[:, None, :] - pos[nbr]
    d2e = jnp.sum(delta * delta, axis=-1)
    s = jnp.where(mask, d2e, 1.0)
    ew = jnp.sqrt(s)
    c = 0.5 * (jnp.cos(ew / CUTOFF * jnp.pi) + 1.0)
    cm = maskf * c

    h = params['emb'][z]

    wk = []
    for i in range(NI):
        w0 = params[f'b{i}_mlp0_w']
        w2 = params[f'b{i}_mlp2_w']
        wk.append(dict(
            w0t=_pad_rows(w0.T, 128),
            b0=params[f'b{i}_mlp0_b'][None, :],
            w2t=w2.T,
            b2=params[f'b{i}_mlp2_b'][None, :],
            w0=_pad_cols(w0, 128),
            w2=w2,
        ))

    saved = []
    for i in range(NI):
        k = wk[i]
        h1 = h @ params[f'b{i}_conv_lin1_w'].T
        g = h1[nbr_flat]
        agg = _edge_fwd(ew, cm, g, k['w0t'], k['b0'], k['w2t'], k['b2'])
        u = h1 + agg
        v = u @ params[f'b{i}_conv_lin2_w'].T + params[f'b{i}_conv_lin2_b']
        sigV = jax.nn.sigmoid(v)
        w_ = _ssp(v)
        h = w_ @ params[f'b{i}_lin_w'].T + params[f'b{i}_lin_b']
        saved.append((g, sigV))

    y1 = h @ params['lin1_w'].T + params['lin1_b']
    sigY = jax.nn.sigmoid(y1)
    y2 = _ssp(y1)
    y3 = y2 @ params['lin2_w'].T + params['lin2_b']
    energy = jnp.sum(y3)

    dy2 = jnp.broadcast_to(params['lin2_w'][0], y2.shape)
    dy1 = dy2 * sigY
    dh = dy1 @ params['lin1_w']

    dew_tot = jnp.zeros((N, MAXNB), jnp.float32)
    for i in range(NI - 1, -1, -1):
        g, sigV = saved[i]
        k = wk[i]
        dw_ = dh @ params[f'b{i}_lin_w']
        dv = dw_ * sigV
        du = dv @ params[f'b{i}_conv_lin2_w']
        dg, dew = _edge_bwd(ew, cm, maskf, g, du,
                            k['w0t'], k['b0'], k['w2t'], k['b2'],
                            k['w0'], k['w2'])
        dew_tot = dew_tot + dew
        dh1 = du + jax.ops.segment_sum(dg, nbr_flat, num_segments=N)
        dh = dh1 @ params[f'b{i}_conv_lin1_w']

    dd2e = maskf * dew_tot * (0.5 / ew)
    ddelta = 2.0 * dd2e[..., None] * delta
    dpos = ddelta.sum(axis=1) - jax.ops.segment_sum(
        ddelta.reshape(N * MAXNB, 3), nbr_flat, num_segments=N)

    return (energy, dpos)


# speedup vs baseline: 1.9591x; 1.9591x over previous
import functools

import jax
import jax.numpy as jnp
from jax import lax
from jax.experimental import pallas as pl
from jax.experimental.pallas import tpu as pltpu
from jax.experimental.pallas import tpu_sc as plsc

HIDDEN = 128
NF = 128
NG = 50
NI = 6
CUTOFF = 5.0
MAXNB = 32
N_ATOMS = 4096

_LOG2 = 0.6931471805599453
_STEP = CUTOFF / (NG - 1)
_GAMMA = 0.5 / _STEP ** 2

_T = 256
_ET = _T * MAXNB

_INTERPRET = False


def _ssp(x):
    return jax.nn.softplus(x) - _LOG2


def _offsets_lanes(shape):
    lane_i = jax.lax.broadcasted_iota(jnp.int32, shape, len(shape) - 1)
    lane = lane_i.astype(jnp.float32)
    off = lane * _STEP
    return jnp.where(lane_i < NG, off, 1e9)


def _edge_fwd_body(ew_ref, cm_ref, g_ref, w0t_ref, b0_ref, w2t_ref, b2_ref,
                   agg_ref):
    ew = ew_ref[...]
    ew3 = jax.lax.broadcast_in_dim(ew, (_T, MAXNB, 128), (0, 1))
    off3 = _offsets_lanes((_T, MAXNB, 128))
    attr = jnp.exp(-_GAMMA * (ew3 - off3) ** 2).reshape(_ET, 128)
    Z = jnp.dot(attr, w0t_ref[...], preferred_element_type=jnp.float32) + b0_ref[...]
    A = _ssp(Z)
    W = jnp.dot(A, w2t_ref[...], preferred_element_type=jnp.float32) + b2_ref[...]
    cm3 = jax.lax.broadcast_in_dim(cm_ref[...], (_T, MAXNB, 128), (0, 1))
    msg = (W * g_ref[...]).reshape(_T, MAXNB, 128) * cm3
    agg_ref[...] = msg.sum(axis=1)


def _edge_bwd_body(ew_ref, cm_ref, mk_ref, g_ref, du_ref,
                   w0t_ref, b0_ref, w2t_ref, b2_ref, w0_ref, w2_ref,
                   dg_ref, dew_ref):
    ew = ew_ref[...]
    ew3 = jax.lax.broadcast_in_dim(ew, (_T, MAXNB, 128), (0, 1))
    off3 = _offsets_lanes((_T, MAXNB, 128))
    attr3 = jnp.exp(-_GAMMA * (ew3 - off3) ** 2)
    attr = attr3.reshape(_ET, 128)
    Z = jnp.dot(attr, w0t_ref[...], preferred_element_type=jnp.float32) + b0_ref[...]
    sigZ = jax.nn.sigmoid(Z)
    A = _ssp(Z)
    W = jnp.dot(A, w2t_ref[...], preferred_element_type=jnp.float32) + b2_ref[...]

    cm3 = jax.lax.broadcast_in_dim(cm_ref[...], (_T, MAXNB, 128), (0, 1))
    cm_e = cm3.reshape(_ET, 128)
    du3 = jax.lax.broadcast_in_dim(du_ref[...], (_T, MAXNB, 128), (0, 2))
    du_e = du3.reshape(_ET, 128)
    g = g_ref[...]

    cdu = cm_e * du_e
    dW = cdu * g
    dg_ref[...] = cdu * W
    dcm = (du_e * W * g).reshape(_T, MAXNB, 128).sum(axis=2)

    dA = jnp.dot(dW, w2_ref[...], preferred_element_type=jnp.float32)
    dZ = dA * sigZ
    dattr = jnp.dot(dZ, w0_ref[...], preferred_element_type=jnp.float32)
    dew_attr = (dattr.reshape(_T, MAXNB, 128) * attr3
                * (-2.0 * _GAMMA) * (ew3 - off3)).sum(axis=2)
    dc = mk_ref[...] * dcm
    dew_ref[...] = dew_attr + dc * (-0.5 * jnp.pi / CUTOFF) * jnp.sin(
        ew * (jnp.pi / CUTOFF))


def _pad_rows(w, rows):
    return jnp.zeros((rows, w.shape[1]), w.dtype).at[:w.shape[0]].set(w)


def _pad_cols(w, cols):
    return jnp.zeros((w.shape[0], cols), w.dtype).at[:, :w.shape[1]].set(w)


@functools.partial(jax.jit, static_argnames=())
def _edge_fwd(ew, cm, g, w0t, b0, w2t, b2):
    N = ew.shape[0]
    grid = N // _T
    return pl.pallas_call(
        _edge_fwd_body,
        grid=(grid,),
        in_specs=[
            pl.BlockSpec((_T, MAXNB), lambda i: (i, 0)),
            pl.BlockSpec((_T, MAXNB), lambda i: (i, 0)),
            pl.BlockSpec((_ET, 128), lambda i: (i, 0)),
            pl.BlockSpec((128, 128), lambda i: (0, 0)),
            pl.BlockSpec((1, 128), lambda i: (0, 0)),
            pl.BlockSpec((128, 128), lambda i: (0, 0)),
            pl.BlockSpec((1, 128), lambda i: (0, 0)),
        ],
        out_specs=pl.BlockSpec((_T, 128), lambda i: (i, 0)),
        out_shape=jax.ShapeDtypeStruct((N, 128), jnp.float32),
        interpret=_INTERPRET,
    )(ew, cm, g, w0t, b0, w2t, b2)


@functools.partial(jax.jit, static_argnames=())
def _edge_bwd(ew, cm, mk, g, du, w0t, b0, w2t, b2, w0, w2):
    N = ew.shape[0]
    grid = N // _T
    return pl.pallas_call(
        _edge_bwd_body,
        grid=(grid,),
        in_specs=[
            pl.BlockSpec((_T, MAXNB), lambda i: (i, 0)),
            pl.BlockSpec((_T, MAXNB), lambda i: (i, 0)),
            pl.BlockSpec((_T, MAXNB), lambda i: (i, 0)),
            pl.BlockSpec((_ET, 128), lambda i: (i, 0)),
            pl.BlockSpec((_T, 128), lambda i: (i, 0)),
            pl.BlockSpec((128, 128), lambda i: (0, 0)),
            pl.BlockSpec((1, 128), lambda i: (0, 0)),
            pl.BlockSpec((128, 128), lambda i: (0, 0)),
            pl.BlockSpec((1, 128), lambda i: (0, 0)),
            pl.BlockSpec((128, 128), lambda i: (0, 0)),
            pl.BlockSpec((128, 128), lambda i: (0, 0)),
        ],
        out_specs=[
            pl.BlockSpec((_ET, 128), lambda i: (i, 0)),
            pl.BlockSpec((_T, MAXNB), lambda i: (i, 0)),
        ],
        out_shape=[
            jax.ShapeDtypeStruct((N * MAXNB, 128), jnp.float32),
            jax.ShapeDtypeStruct((N, MAXNB), jnp.float32),
        ],
        interpret=_INTERPRET,
    )(ew, cm, mk, g, du, w0t, b0, w2t, b2, w0, w2)


_NC, _NS = 2, 16
_NW = _NC * _NS
_E = N_ATOMS * MAXNB
_EPW = _E // _NW
_GCH = 512
_NCH = _EPW // _GCH


def _sc_gather_body(table_hbm, idx_hbm, out_hbm, idx_v, rows_v, sem):
    wid = lax.axis_index("s") * _NC + lax.axis_index("c")
    base = wid * _EPW
    for i in range(_NCH):
        off = base + i * _GCH
        pltpu.sync_copy(idx_hbm.at[pl.ds(off, _GCH)], idx_v)
        pltpu.async_copy(table_hbm.at[idx_v], rows_v, sem).wait()
        pltpu.sync_copy(rows_v, out_hbm.at[pl.ds(off, _GCH)])


def _sc_gather(table, idx):
    mesh = plsc.VectorSubcoreMesh(core_axis_name="c", subcore_axis_name="s")
    return pl.kernel(
        _sc_gather_body,
        out_type=jax.ShapeDtypeStruct((idx.shape[0], 128), jnp.float32),
        mesh=mesh,
        scratch_types=[
            pltpu.VMEM((_GCH,), jnp.int32),
            pltpu.VMEM((_GCH, 128), jnp.float32),
            pltpu.SemaphoreType.DMA,
        ],
    )(table, idx)


def _sc_scatter_body(idx_hbm, val_hbm, zeros_hbm, out_hbm, idx_v, rows_v,
                     acc_sh, sem):
    c = lax.axis_index("c")
    s = lax.axis_index("s")
    wid = s * _NC + c

    @pl.when(s == 0)
    def _():
        pltpu.sync_copy(zeros_hbm, acc_sh)

    plsc.subcore_barrier()
    base = wid * _EPW
    for i in range(_NCH):
        off = base + i * _GCH
        pltpu.sync_copy(idx_hbm.at[pl.ds(off, _GCH)], idx_v)
        pltpu.sync_copy(val_hbm.at[pl.ds(off, _GCH)], rows_v)
        pltpu.sync_copy(rows_v, acc_sh.at[idx_v], add=True)
    plsc.subcore_barrier()
    rps = N_ATOMS // _NS
    pltpu.sync_copy(acc_sh.at[pl.ds(s * rps, rps)],
                    out_hbm.at[pl.ds(c * N_ATOMS + s * rps, rps)])


def _sc_scatter(idx, val, zeros):
    mesh = plsc.VectorSubcoreMesh(core_axis_name="c", subcore_axis_name="s")
    return pl.kernel(
        _sc_scatter_body,
        out_type=jax.ShapeDtypeStruct((2 * N_ATOMS, 128), jnp.float32),
        mesh=mesh,
        scratch_types=[
            pltpu.VMEM((_GCH,), jnp.int32),
            pltpu.VMEM((_GCH, 128), jnp.float32),
            pltpu.VMEM_SHARED((N_ATOMS, 128), jnp.float32),
            pltpu.SemaphoreType.DMA,
        ],
    )(idx, val, zeros)


_TR = 256


def _topk_body(dist_ref, dsel_ref, order_ref):
    row = dist_ref[...]
    n = row.shape[1]
    lane = jax.lax.broadcasted_iota(jnp.int32, (_TR, n), 1)
    big_i = jnp.int32(2 ** 30)
    dsel_cols = []
    order_cols = []
    for k in range(MAXNB):
        m = jnp.min(row, axis=1, keepdims=True)
        hit = row == m
        idx = jnp.min(jnp.where(hit, lane, big_i), axis=1, keepdims=True)
        dsel_cols.append(m)
        order_cols.append(idx)
        row = jnp.where(lane == idx, jnp.inf, row)
    dsel_ref[...] = jnp.concatenate(dsel_cols, axis=1)
    order_ref[...] = jnp.concatenate(order_cols, axis=1)


@functools.partial(jax.jit, static_argnames=())
def _topk(dist):
    N = dist.shape[0]
    return pl.pallas_call(
        _topk_body,
        grid=(N // _TR,),
        in_specs=[pl.BlockSpec((_TR, N), lambda i: (i, 0))],
        out_specs=[
            pl.BlockSpec((_TR, MAXNB), lambda i: (i, 0)),
            pl.BlockSpec((_TR, MAXNB), lambda i: (i, 0)),
        ],
        out_shape=[
            jax.ShapeDtypeStruct((N, MAXNB), jnp.float32),
            jax.ShapeDtypeStruct((N, MAXNB), jnp.int32),
        ],
        interpret=_INTERPRET,
    )(dist)


def _graph(pos):
    N = pos.shape[0]
    sq = jnp.sum(pos ** 2, axis=-1)
    d2 = sq[:, None] + sq[None, :] - 2.0 * pos @ pos.T
    d2 = jnp.maximum(d2, 0.0)
    dist = jnp.sqrt(d2)
    dist = jnp.where(jnp.eye(N, dtype=bool), jnp.inf, dist)
    d_sel, order = _topk(dist)
    mask = d_sel < CUTOFF
    centers = jnp.arange(N, dtype=order.dtype)[:, None]
    nbr = jnp.where(mask, order, centers).astype(jnp.int32)
    return nbr, mask


def kernel(z, pos, params):
    N = pos.shape[0]
    nbr, mask = _graph(pos)
    nbr_flat = nbr.reshape(-1)
    maskf = mask.astype(jnp.float32)

    delta = pos[:, None, :] - pos[nbr]
    d2e = jnp.sum(delta * delta, axis=-1)
    s = jnp.where(mask, d2e, 1.0)
    ew = jnp.sqrt(s)
    c = 0.5 * (jnp.cos(ew / CUTOFF * jnp.pi) + 1.0)
    cm = maskf * c

    h = params['emb'][z]

    wk = []
    for i in range(NI):
        w0 = params[f'b{i}_mlp0_w']
        w2 = params[f'b{i}_mlp2_w']
        wk.append(dict(
            w0t=_pad_rows(w0.T, 128),
            b0=params[f'b{i}_mlp0_b'][None, :],
            w2t=w2.T,
            b2=params[f'b{i}_mlp2_b'][None, :],
            w0=_pad_cols(w0, 128),
            w2=w2,
        ))

    saved = []
    for i in range(NI):
        k = wk[i]
        h1 = h @ params[f'b{i}_conv_lin1_w'].T
        g = _sc_gather(h1, nbr_flat)
        agg = _edge_fwd(ew, cm, g, k['w0t'], k['b0'], k['w2t'], k['b2'])
        u = h1 + agg
        v = u @ params[f'b{i}_conv_lin2_w'].T + params[f'b{i}_conv_lin2_b']
        sigV = jax.nn.sigmoid(v)
        w_ = _ssp(v)
        h = w_ @ params[f'b{i}_lin_w'].T + params[f'b{i}_lin_b']
        saved.append((g, sigV))

    y1 = h @ params['lin1_w'].T + params['lin1_b']
    sigY = jax.nn.sigmoid(y1)
    y2 = _ssp(y1)
    y3 = y2 @ params['lin2_w'].T + params['lin2_b']
    energy = jnp.sum(y3)

    zeros_nf = jnp.zeros((N, 128), jnp.float32)
    dy2 = jnp.broadcast_to(params['lin2_w'][0], y2.shape)
    dy1 = dy2 * sigY
    dh = dy1 @ params['lin1_w']

    dew_tot = jnp.zeros((N, MAXNB), jnp.float32)
    for i in range(NI - 1, -1, -1):
        g, sigV = saved[i]
        k = wk[i]
        dw_ = dh @ params[f'b{i}_lin_w']
        dv = dw_ * sigV
        du = dv @ params[f'b{i}_conv_lin2_w']
        dg, dew = _edge_bwd(ew, cm, maskf, g, du,
                            k['w0t'], k['b0'], k['w2t'], k['b2'],
                            k['w0'], k['w2'])
        dew_tot = dew_tot + dew
        sc = _sc_scatter(nbr_flat, dg, zeros_nf)
        dh1 = du + sc[:N] + sc[N:]
        dh = dh1 @ params[f'b{i}_conv_lin1_w']

    dd2e = maskf * dew_tot * (0.5 / ew)
    ddelta = 2.0 * dd2e[..., None] * delta
    dpos = ddelta.sum(axis=1) - jax.ops.segment_sum(
        ddelta.reshape(N * MAXNB, 3), nbr_flat, num_segments=N)

    return (energy, dpos)


# speedup vs baseline: 2.2134x; 1.1298x over previous
import functools

import jax
import jax.numpy as jnp
from jax import lax
from jax.experimental import pallas as pl
from jax.experimental.pallas import tpu as pltpu
from jax.experimental.pallas import tpu_sc as plsc

HIDDEN = 128
NF = 128
NG = 50
NI = 6
CUTOFF = 5.0
MAXNB = 32
N_ATOMS = 4096

_LOG2 = 0.6931471805599453
_STEP = CUTOFF / (NG - 1)
_GAMMA = 0.5 / _STEP ** 2

_T = 256
_ET = _T * MAXNB

_INTERPRET = False


def _ssp(x):
    return jax.nn.softplus(x) - _LOG2


def _offsets_lanes(shape):
    lane_i = jax.lax.broadcasted_iota(jnp.int32, shape, len(shape) - 1)
    lane = lane_i.astype(jnp.float32)
    off = lane * _STEP
    return jnp.where(lane_i < NG, off, 1e9)


def _edge_fwd_body(ew_ref, cm_ref, g_ref, w0t_ref, b0_ref, w2t_ref, b2_ref,
                   agg_ref):
    ew = ew_ref[...]
    ew3 = jax.lax.broadcast_in_dim(ew, (_T, MAXNB, 128), (0, 1))
    off3 = _offsets_lanes((_T, MAXNB, 128))
    attr = jnp.exp(-_GAMMA * (ew3 - off3) ** 2).reshape(_ET, 128)
    Z = jnp.dot(attr, w0t_ref[...], preferred_element_type=jnp.float32) + b0_ref[...]
    A = _ssp(Z)
    W = jnp.dot(A, w2t_ref[...], preferred_element_type=jnp.float32) + b2_ref[...]
    cm3 = jax.lax.broadcast_in_dim(cm_ref[...], (_T, MAXNB, 128), (0, 1))
    msg = (W * g_ref[...]).reshape(_T, MAXNB, 128) * cm3
    agg_ref[...] = msg.sum(axis=1)


def _edge_bwd_body(ew_ref, cm_ref, mk_ref, g_ref, du_ref,
                   w0t_ref, b0_ref, w2t_ref, b2_ref, w0_ref, w2_ref,
                   dg_ref, dew_ref):
    ew = ew_ref[...]
    ew3 = jax.lax.broadcast_in_dim(ew, (_T, MAXNB, 128), (0, 1))
    off3 = _offsets_lanes((_T, MAXNB, 128))
    attr3 = jnp.exp(-_GAMMA * (ew3 - off3) ** 2)
    attr = attr3.reshape(_ET, 128)
    Z = jnp.dot(attr, w0t_ref[...], preferred_element_type=jnp.float32) + b0_ref[...]
    sigZ = jax.nn.sigmoid(Z)
    A = _ssp(Z)
    W = jnp.dot(A, w2t_ref[...], preferred_element_type=jnp.float32) + b2_ref[...]

    cm3 = jax.lax.broadcast_in_dim(cm_ref[...], (_T, MAXNB, 128), (0, 1))
    cm_e = cm3.reshape(_ET, 128)
    du3 = jax.lax.broadcast_in_dim(du_ref[...], (_T, MAXNB, 128), (0, 2))
    du_e = du3.reshape(_ET, 128)
    g = g_ref[...]

    cdu = cm_e * du_e
    dW = cdu * g
    dg_ref[...] = cdu * W
    dcm = (du_e * W * g).reshape(_T, MAXNB, 128).sum(axis=2)

    dA = jnp.dot(dW, w2_ref[...], preferred_element_type=jnp.float32)
    dZ = dA * sigZ
    dattr = jnp.dot(dZ, w0_ref[...], preferred_element_type=jnp.float32)
    dew_attr = (dattr.reshape(_T, MAXNB, 128) * attr3
                * (-2.0 * _GAMMA) * (ew3 - off3)).sum(axis=2)
    dc = mk_ref[...] * dcm
    dew_ref[...] = dew_attr + dc * (-0.5 * jnp.pi / CUTOFF) * jnp.sin(
        ew * (jnp.pi / CUTOFF))


def _pad_rows(w, rows):
    return jnp.zeros((rows, w.shape[1]), w.dtype).at[:w.shape[0]].set(w)


def _pad_cols(w, cols):
    return jnp.zeros((w.shape[0], cols), w.dtype).at[:, :w.shape[1]].set(w)


@functools.partial(jax.jit, static_argnames=())
def _edge_fwd(ew, cm, g, w0t, b0, w2t, b2):
    N = ew.shape[0]
    grid = N // _T
    return pl.pallas_call(
        _edge_fwd_body,
        grid=(grid,),
        in_specs=[
            pl.BlockSpec((_T, MAXNB), lambda i: (i, 0)),
            pl.BlockSpec((_T, MAXNB), lambda i: (i, 0)),
            pl.BlockSpec((_ET, 128), lambda i: (i, 0)),
            pl.BlockSpec((128, 128), lambda i: (0, 0)),
            pl.BlockSpec((1, 128), lambda i: (0, 0)),
            pl.BlockSpec((128, 128), lambda i: (0, 0)),
            pl.BlockSpec((1, 128), lambda i: (0, 0)),
        ],
        out_specs=pl.BlockSpec((_T, 128), lambda i: (i, 0)),
        out_shape=jax.ShapeDtypeStruct((N, 128), jnp.float32),
        interpret=_INTERPRET,
    )(ew, cm, g, w0t, b0, w2t, b2)


@functools.partial(jax.jit, static_argnames=())
def _edge_bwd(ew, cm, mk, g, du, w0t, b0, w2t, b2, w0, w2):
    N = ew.shape[0]
    grid = N // _T
    return pl.pallas_call(
        _edge_bwd_body,
        grid=(grid,),
        in_specs=[
            pl.BlockSpec((_T, MAXNB), lambda i: (i, 0)),
            pl.BlockSpec((_T, MAXNB), lambda i: (i, 0)),
            pl.BlockSpec((_T, MAXNB), lambda i: (i, 0)),
            pl.BlockSpec((_ET, 128), lambda i: (i, 0)),
            pl.BlockSpec((_T, 128), lambda i: (i, 0)),
            pl.BlockSpec((128, 128), lambda i: (0, 0)),
            pl.BlockSpec((1, 128), lambda i: (0, 0)),
            pl.BlockSpec((128, 128), lambda i: (0, 0)),
            pl.BlockSpec((1, 128), lambda i: (0, 0)),
            pl.BlockSpec((128, 128), lambda i: (0, 0)),
            pl.BlockSpec((128, 128), lambda i: (0, 0)),
        ],
        out_specs=[
            pl.BlockSpec((_ET, 128), lambda i: (i, 0)),
            pl.BlockSpec((_T, MAXNB), lambda i: (i, 0)),
        ],
        out_shape=[
            jax.ShapeDtypeStruct((N * MAXNB, 128), jnp.float32),
            jax.ShapeDtypeStruct((N, MAXNB), jnp.float32),
        ],
        interpret=_INTERPRET,
    )(ew, cm, mk, g, du, w0t, b0, w2t, b2, w0, w2)


_NC, _NS = 2, 16
_NW = _NC * _NS
_E = N_ATOMS * MAXNB
_EPW = _E // _NW
_GCH = 512
_NCH = _EPW // _GCH


def _sc_gather_body(w, table_hbm, idx_hbm, out_hbm, idx_v, rows_v, sem):
    wid = lax.axis_index("s") * _NC + lax.axis_index("c")
    base = wid * _EPW
    for i in range(_NCH):
        off = base + i * _GCH
        pltpu.sync_copy(idx_hbm.at[pl.ds(off, _GCH)], idx_v)
        pltpu.async_copy(table_hbm.at[idx_v], rows_v, sem).wait()
        pltpu.sync_copy(rows_v, out_hbm.at[pl.ds(off, _GCH)])


def _sc_gather(table, idx):
    w = table.shape[1]
    mesh = plsc.VectorSubcoreMesh(core_axis_name="c", subcore_axis_name="s")
    return pl.kernel(
        functools.partial(_sc_gather_body, w),
        out_type=jax.ShapeDtypeStruct((idx.shape[0], w), jnp.float32),
        mesh=mesh,
        scratch_types=[
            pltpu.VMEM((_GCH,), jnp.int32),
            pltpu.VMEM((_GCH, w), jnp.float32),
            pltpu.SemaphoreType.DMA,
        ],
    )(table, idx)


def _sc_scatter_body(w, idx_hbm, val_hbm, zeros_hbm, out_hbm, idx_v, rows_v,
                     acc_sh, sem):
    c = lax.axis_index("c")
    s = lax.axis_index("s")
    wid = s * _NC + c

    @pl.when(s == 0)
    def _():
        pltpu.sync_copy(zeros_hbm, acc_sh)

    plsc.subcore_barrier()
    base = wid * _EPW
    for i in range(_NCH):
        off = base + i * _GCH
        pltpu.sync_copy(idx_hbm.at[pl.ds(off, _GCH)], idx_v)
        pltpu.sync_copy(val_hbm.at[pl.ds(off, _GCH)], rows_v)
        pltpu.sync_copy(rows_v, acc_sh.at[idx_v], add=True)
    plsc.subcore_barrier()
    rps = N_ATOMS // _NS
    pltpu.sync_copy(acc_sh.at[pl.ds(s * rps, rps)],
                    out_hbm.at[pl.ds(c * N_ATOMS + s * rps, rps)])


def _sc_scatter(idx, val, zeros):
    w = val.shape[1]
    mesh = plsc.VectorSubcoreMesh(core_axis_name="c", subcore_axis_name="s")
    return pl.kernel(
        functools.partial(_sc_scatter_body, w),
        out_type=jax.ShapeDtypeStruct((2 * N_ATOMS, w), jnp.float32),
        mesh=mesh,
        scratch_types=[
            pltpu.VMEM((_GCH,), jnp.int32),
            pltpu.VMEM((_GCH, w), jnp.float32),
            pltpu.VMEM_SHARED((N_ATOMS, w), jnp.float32),
            pltpu.SemaphoreType.DMA,
        ],
    )(idx, val, zeros)


_TR = 256


def _topk_body(dist_ref, dsel_ref, order_ref):
    row = dist_ref[...]
    n = row.shape[1]
    lane = jax.lax.broadcasted_iota(jnp.int32, (_TR, n), 1)
    big_i = jnp.int32(2 ** 30)
    dsel_cols = []
    order_cols = []
    for k in range(MAXNB):
        m = jnp.min(row, axis=1, keepdims=True)
        hit = row == m
        idx = jnp.min(jnp.where(hit, lane, big_i), axis=1, keepdims=True)
        dsel_cols.append(m)
        order_cols.append(idx)
        row = jnp.where(lane == idx, jnp.inf, row)
    dsel_ref[...] = jnp.concatenate(dsel_cols, axis=1)
    order_ref[...] = jnp.concatenate(order_cols, axis=1)


@functools.partial(jax.jit, static_argnames=())
def _topk(dist):
    N = dist.shape[0]
    return pl.pallas_call(
        _topk_body,
        grid=(N // _TR,),
        in_specs=[pl.BlockSpec((_TR, N), lambda i: (i, 0))],
        out_specs=[
            pl.BlockSpec((_TR, MAXNB), lambda i: (i, 0)),
            pl.BlockSpec((_TR, MAXNB), lambda i: (i, 0)),
        ],
        out_shape=[
            jax.ShapeDtypeStruct((N, MAXNB), jnp.float32),
            jax.ShapeDtypeStruct((N, MAXNB), jnp.int32),
        ],
        interpret=_INTERPRET,
    )(dist)


def _graph(pos):
    N = pos.shape[0]
    sq = jnp.sum(pos ** 2, axis=-1)
    d2 = sq[:, None] + sq[None, :] - 2.0 * pos @ pos.T
    d2 = jnp.maximum(d2, 0.0)
    dist = jnp.sqrt(d2)
    dist = jnp.where(jnp.eye(N, dtype=bool), jnp.inf, dist)
    d_sel, order = _topk(dist)
    mask = d_sel < CUTOFF
    centers = jnp.arange(N, dtype=order.dtype)[:, None]
    nbr = jnp.where(mask, order, centers).astype(jnp.int32)
    return nbr, mask


def kernel(z, pos, params):
    N = pos.shape[0]
    nbr, mask = _graph(pos)
    nbr_flat = nbr.reshape(-1)
    maskf = mask.astype(jnp.float32)

    pos128 = jnp.concatenate([pos, jnp.zeros((N, 125), jnp.float32)], axis=1)
    psrc = _sc_gather(pos128, nbr_flat)[:, :3].reshape(N, MAXNB, 3)
    delta = pos[:, None, :] - psrc
    d2e = jnp.sum(delta * delta, axis=-1)
    s = jnp.where(mask, d2e, 1.0)
    ew = jnp.sqrt(s)
    c = 0.5 * (jnp.cos(ew / CUTOFF * jnp.pi) + 1.0)
    cm = maskf * c

    h = params['emb'][z]

    wk = []
    for i in range(NI):
        w0 = params[f'b{i}_mlp0_w']
        w2 = params[f'b{i}_mlp2_w']
        wk.append(dict(
            w0t=_pad_rows(w0.T, 128),
            b0=params[f'b{i}_mlp0_b'][None, :],
            w2t=w2.T,
            b2=params[f'b{i}_mlp2_b'][None, :],
            w0=_pad_cols(w0, 128),
            w2=w2,
        ))

    saved = []
    for i in range(NI):
        k = wk[i]
        h1 = h @ params[f'b{i}_conv_lin1_w'].T
        g = _sc_gather(h1, nbr_flat)
        agg = _edge_fwd(ew, cm, g, k['w0t'], k['b0'], k['w2t'], k['b2'])
        u = h1 + agg
        v = u @ params[f'b{i}_conv_lin2_w'].T + params[f'b{i}_conv_lin2_b']
        sigV = jax.nn.sigmoid(v)
        w_ = _ssp(v)
        h = w_ @ params[f'b{i}_lin_w'].T + params[f'b{i}_lin_b']
        saved.append((g, sigV))

    y1 = h @ params['lin1_w'].T + params['lin1_b']
    sigY = jax.nn.sigmoid(y1)
    y2 = _ssp(y1)
    y3 = y2 @ params['lin2_w'].T + params['lin2_b']
    energy = jnp.sum(y3)

    zeros_nf = jnp.zeros((N, 128), jnp.float32)
    dy2 = jnp.broadcast_to(params['lin2_w'][0], y2.shape)
    dy1 = dy2 * sigY
    dh = dy1 @ params['lin1_w']

    dew_tot = jnp.zeros((N, MAXNB), jnp.float32)
    for i in range(NI - 1, -1, -1):
        g, sigV = saved[i]
        k = wk[i]
        dw_ = dh @ params[f'b{i}_lin_w']
        dv = dw_ * sigV
        du = dv @ params[f'b{i}_conv_lin2_w']
        dg, dew = _edge_bwd(ew, cm, maskf, g, du,
                            k['w0t'], k['b0'], k['w2t'], k['b2'],
                            k['w0'], k['w2'])
        dew_tot = dew_tot + dew
        sc = _sc_scatter(nbr_flat, dg, zeros_nf)
        dh1 = du + sc[:N] + sc[N:]
        dh = dh1 @ params[f'b{i}_conv_lin1_w']

    dd2e = maskf * dew_tot * (0.5 / ew)
    ddelta = 2.0 * dd2e[..., None] * delta
    dd128 = jnp.concatenate([ddelta.reshape(N * MAXNB, 3),
                             jnp.zeros((N * MAXNB, 125), jnp.float32)], axis=1)
    sc3 = _sc_scatter(nbr_flat, dd128, zeros_nf)
    dpos = ddelta.sum(axis=1) - (sc3[:N, :3] + sc3[N:, :3])

    return (energy, dpos)


# speedup vs baseline: 2.2450x; 1.0143x over previous
import functools

import jax
import jax.numpy as jnp
from jax import lax
from jax.experimental import pallas as pl
from jax.experimental.pallas import tpu as pltpu
from jax.experimental.pallas import tpu_sc as plsc

HIDDEN = 128
NF = 128
NG = 50
NI = 6
CUTOFF = 5.0
MAXNB = 32
N_ATOMS = 4096

_LOG2 = 0.6931471805599453
_STEP = CUTOFF / (NG - 1)
_GAMMA = 0.5 / _STEP ** 2

_T = 256
_ET = _T * MAXNB

_INTERPRET = False


def _ssp(x):
    return jax.nn.softplus(x) - _LOG2


def _offsets_lanes(shape):
    lane_i = jax.lax.broadcasted_iota(jnp.int32, shape, len(shape) - 1)
    lane = lane_i.astype(jnp.float32)
    off = lane * _STEP
    return jnp.where(lane_i < NG, off, 1e9)


def _edge_fwd_body(ew_ref, cm_ref, g_ref, w0t_ref, b0_ref, w2t_ref, b2_ref,
                   agg_ref):
    ew = ew_ref[...]
    ew3 = jax.lax.broadcast_in_dim(ew, (_T, MAXNB, 128), (0, 1))
    off3 = _offsets_lanes((_T, MAXNB, 128))
    attr = jnp.exp(-_GAMMA * (ew3 - off3) ** 2).reshape(_ET, 128)
    Z = jnp.dot(attr, w0t_ref[...], preferred_element_type=jnp.float32) + b0_ref[...]
    A = _ssp(Z)
    W = jnp.dot(A, w2t_ref[...], preferred_element_type=jnp.float32) + b2_ref[...]
    cm3 = jax.lax.broadcast_in_dim(cm_ref[...], (_T, MAXNB, 128), (0, 1))
    msg = (W * g_ref[...]).reshape(_T, MAXNB, 128) * cm3
    agg_ref[...] = msg.sum(axis=1)


def _edge_bwd_body(ew_ref, cm_ref, mk_ref, g_ref, du_ref,
                   w0t_ref, b0_ref, w2t_ref, b2_ref, w0_ref, w2_ref,
                   dg_ref, dew_ref):
    ew = ew_ref[...]
    ew3 = jax.lax.broadcast_in_dim(ew, (_T, MAXNB, 128), (0, 1))
    off3 = _offsets_lanes((_T, MAXNB, 128))
    attr3 = jnp.exp(-_GAMMA * (ew3 - off3) ** 2)
    attr = attr3.reshape(_ET, 128)
    Z = jnp.dot(attr, w0t_ref[...], preferred_element_type=jnp.float32) + b0_ref[...]
    sigZ = jax.nn.sigmoid(Z)
    A = _ssp(Z)
    W = jnp.dot(A, w2t_ref[...], preferred_element_type=jnp.float32) + b2_ref[...]

    cm3 = jax.lax.broadcast_in_dim(cm_ref[...], (_T, MAXNB, 128), (0, 1))
    cm_e = cm3.reshape(_ET, 128)
    du3 = jax.lax.broadcast_in_dim(du_ref[...], (_T, MAXNB, 128), (0, 2))
    du_e = du3.reshape(_ET, 128)
    g = g_ref[...]

    cdu = cm_e * du_e
    dW = cdu * g
    dg_ref[...] = cdu * W
    dcm = (du_e * W * g).reshape(_T, MAXNB, 128).sum(axis=2)

    dA = jnp.dot(dW, w2_ref[...], preferred_element_type=jnp.float32)
    dZ = dA * sigZ
    dattr = jnp.dot(dZ, w0_ref[...], preferred_element_type=jnp.float32)
    dew_attr = (dattr.reshape(_T, MAXNB, 128) * attr3
                * (-2.0 * _GAMMA) * (ew3 - off3)).sum(axis=2)
    dc = mk_ref[...] * dcm
    dew_ref[...] = dew_attr + dc * (-0.5 * jnp.pi / CUTOFF) * jnp.sin(
        ew * (jnp.pi / CUTOFF))


def _pad_rows(w, rows):
    return jnp.zeros((rows, w.shape[1]), w.dtype).at[:w.shape[0]].set(w)


def _pad_cols(w, cols):
    return jnp.zeros((w.shape[0], cols), w.dtype).at[:, :w.shape[1]].set(w)


@functools.partial(jax.jit, static_argnames=())
def _edge_fwd(ew, cm, g, w0t, b0, w2t, b2):
    N = ew.shape[0]
    grid = N // _T
    return pl.pallas_call(
        _edge_fwd_body,
        grid=(grid,),
        in_specs=[
            pl.BlockSpec((_T, MAXNB), lambda i: (i, 0)),
            pl.BlockSpec((_T, MAXNB), lambda i: (i, 0)),
            pl.BlockSpec((_ET, 128), lambda i: (i, 0)),
            pl.BlockSpec((128, 128), lambda i: (0, 0)),
            pl.BlockSpec((1, 128), lambda i: (0, 0)),
            pl.BlockSpec((128, 128), lambda i: (0, 0)),
            pl.BlockSpec((1, 128), lambda i: (0, 0)),
        ],
        out_specs=pl.BlockSpec((_T, 128), lambda i: (i, 0)),
        out_shape=jax.ShapeDtypeStruct((N, 128), jnp.float32),
        interpret=_INTERPRET,
    )(ew, cm, g, w0t, b0, w2t, b2)


@functools.partial(jax.jit, static_argnames=())
def _edge_bwd(ew, cm, mk, g, du, w0t, b0, w2t, b2, w0, w2):
    N = ew.shape[0]
    grid = N // _T
    return pl.pallas_call(
        _edge_bwd_body,
        grid=(grid,),
        in_specs=[
            pl.BlockSpec((_T, MAXNB), lambda i: (i, 0)),
            pl.BlockSpec((_T, MAXNB), lambda i: (i, 0)),
            pl.BlockSpec((_T, MAXNB), lambda i: (i, 0)),
            pl.BlockSpec((_ET, 128), lambda i: (i, 0)),
            pl.BlockSpec((_T, 128), lambda i: (i, 0)),
            pl.BlockSpec((128, 128), lambda i: (0, 0)),
            pl.BlockSpec((1, 128), lambda i: (0, 0)),
            pl.BlockSpec((128, 128), lambda i: (0, 0)),
            pl.BlockSpec((1, 128), lambda i: (0, 0)),
            pl.BlockSpec((128, 128), lambda i: (0, 0)),
            pl.BlockSpec((128, 128), lambda i: (0, 0)),
        ],
        out_specs=[
            pl.BlockSpec((_ET, 128), lambda i: (i, 0)),
            pl.BlockSpec((_T, MAXNB), lambda i: (i, 0)),
        ],
        out_shape=[
            jax.ShapeDtypeStruct((N * MAXNB, 128), jnp.float32),
            jax.ShapeDtypeStruct((N, MAXNB), jnp.float32),
        ],
        interpret=_INTERPRET,
    )(ew, cm, mk, g, du, w0t, b0, w2t, b2, w0, w2)


_NC, _NS = 2, 16
_NW = _NC * _NS
_E = N_ATOMS * MAXNB
_EPW = _E // _NW
_GCH = 256
_NCH = _EPW // _GCH


def _sc_gather_body(w, table_hbm, idx_hbm, out_hbm,
                    idx0, idx1, rows0, rows1, sg0, sg1, so0, so1):
    idxv = [idx0, idx1]
    rows = [rows0, rows1]
    sg = [sg0, sg1]
    so = [so0, so1]
    wid = lax.axis_index("s") * _NC + lax.axis_index("c")
    base = wid * _EPW
    gh = [None, None]
    oh = [None, None]
    pltpu.sync_copy(idx_hbm.at[pl.ds(base, _GCH)], idxv[0])
    gh[0] = pltpu.async_copy(table_hbm.at[idxv[0]], rows[0], sg[0])
    for i in range(_NCH):
        b = i % 2
        off = base + i * _GCH
        gh[b].wait()
        oh[b] = pltpu.async_copy(rows[b], out_hbm.at[pl.ds(off, _GCH)], so[b])
        ni = i + 1
        if ni < _NCH:
            nb = ni % 2
            pltpu.sync_copy(idx_hbm.at[pl.ds(base + ni * _GCH, _GCH)], idxv[nb])
            if oh[nb] is not None:
                oh[nb].wait()
            gh[nb] = pltpu.async_copy(table_hbm.at[idxv[nb]], rows[nb], sg[nb])
    oh[(_NCH - 1) % 2].wait()
    if _NCH >= 2:
        oh[_NCH % 2].wait()


def _sc_gather(table, idx):
    w = table.shape[1]
    mesh = plsc.VectorSubcoreMesh(core_axis_name="c", subcore_axis_name="s")
    return pl.kernel(
        functools.partial(_sc_gather_body, w),
        out_type=jax.ShapeDtypeStruct((idx.shape[0], w), jnp.float32),
        mesh=mesh,
        scratch_types=[
            pltpu.VMEM((_GCH,), jnp.int32),
            pltpu.VMEM((_GCH,), jnp.int32),
            pltpu.VMEM((_GCH, w), jnp.float32),
            pltpu.VMEM((_GCH, w), jnp.float32),
            pltpu.SemaphoreType.DMA,
            pltpu.SemaphoreType.DMA,
            pltpu.SemaphoreType.DMA,
            pltpu.SemaphoreType.DMA,
        ],
    )(table, idx)


def _sc_scatter_body(w, idx_hbm, val_hbm, zeros_hbm, out_hbm,
                     idx0, idx1, rows0, rows1, acc_sh, sv0, sv1):
    idxv = [idx0, idx1]
    rows = [rows0, rows1]
    sv = [sv0, sv1]
    c = lax.axis_index("c")
    s = lax.axis_index("s")
    wid = s * _NC + c

    @pl.when(s == 0)
    def _():
        pltpu.sync_copy(zeros_hbm, acc_sh)

    plsc.subcore_barrier()
    base = wid * _EPW
    vh = [None, None]
    pltpu.sync_copy(idx_hbm.at[pl.ds(base, _GCH)], idxv[0])
    vh[0] = pltpu.async_copy(val_hbm.at[pl.ds(base, _GCH)], rows[0], sv[0])
    for i in range(_NCH):
        b = i % 2
        ni = i + 1
        if ni < _NCH:
            nb = ni % 2
            noff = base + ni * _GCH
            pltpu.sync_copy(idx_hbm.at[pl.ds(noff, _GCH)], idxv[nb])
            vh[nb] = pltpu.async_copy(val_hbm.at[pl.ds(noff, _GCH)], rows[nb],
                                      sv[nb])
        vh[b].wait()
        pltpu.sync_copy(rows[b], acc_sh.at[idxv[b]], add=True)
    plsc.subcore_barrier()
    rps = N_ATOMS // _NS
    pltpu.sync_copy(acc_sh.at[pl.ds(s * rps, rps)],
                    out_hbm.at[pl.ds(c * N_ATOMS + s * rps, rps)])


def _sc_scatter(idx, val, zeros):
    w = val.shape[1]
    mesh = plsc.VectorSubcoreMesh(core_axis_name="c", subcore_axis_name="s")
    return pl.kernel(
        functools.partial(_sc_scatter_body, w),
        out_type=jax.ShapeDtypeStruct((2 * N_ATOMS, w), jnp.float32),
        mesh=mesh,
        scratch_types=[
            pltpu.VMEM((_GCH,), jnp.int32),
            pltpu.VMEM((_GCH,), jnp.int32),
            pltpu.VMEM((_GCH, w), jnp.float32),
            pltpu.VMEM((_GCH, w), jnp.float32),
            pltpu.VMEM_SHARED((N_ATOMS, w), jnp.float32),
            pltpu.SemaphoreType.DMA,
            pltpu.SemaphoreType.DMA,
        ],
    )(idx, val, zeros)


_TR = 256


def _topk_body(dist_ref, dsel_ref, order_ref):
    row = dist_ref[...]
    n = row.shape[1]
    lane = jax.lax.broadcasted_iota(jnp.int32, (_TR, n), 1)
    big_i = jnp.int32(2 ** 30)
    dsel_cols = []
    order_cols = []
    for k in range(MAXNB):
        m = jnp.min(row, axis=1, keepdims=True)
        hit = row == m
        idx = jnp.min(jnp.where(hit, lane, big_i), axis=1, keepdims=True)
        dsel_cols.append(m)
        order_cols.append(idx)
        row = jnp.where(lane == idx, jnp.inf, row)
    dsel_ref[...] = jnp.concatenate(dsel_cols, axis=1)
    order_ref[...] = jnp.concatenate(order_cols, axis=1)


@functools.partial(jax.jit, static_argnames=())
def _topk(dist):
    N = dist.shape[0]
    return pl.pallas_call(
        _topk_body,
        grid=(N // _TR,),
        in_specs=[pl.BlockSpec((_TR, N), lambda i: (i, 0))],
        out_specs=[
            pl.BlockSpec((_TR, MAXNB), lambda i: (i, 0)),
            pl.BlockSpec((_TR, MAXNB), lambda i: (i, 0)),
        ],
        out_shape=[
            jax.ShapeDtypeStruct((N, MAXNB), jnp.float32),
            jax.ShapeDtypeStruct((N, MAXNB), jnp.int32),
        ],
        interpret=_INTERPRET,
    )(dist)


def _graph(pos):
    N = pos.shape[0]
    sq = jnp.sum(pos ** 2, axis=-1)
    d2 = sq[:, None] + sq[None, :] - 2.0 * pos @ pos.T
    d2 = jnp.maximum(d2, 0.0)
    dist = jnp.sqrt(d2)
    dist = jnp.where(jnp.eye(N, dtype=bool), jnp.inf, dist)
    d_sel, order = _topk(dist)
    mask = d_sel < CUTOFF
    centers = jnp.arange(N, dtype=order.dtype)[:, None]
    nbr = jnp.where(mask, order, centers).astype(jnp.int32)
    return nbr, mask


def kernel(z, pos, params):
    N = pos.shape[0]
    nbr, mask = _graph(pos)
    nbr_flat = nbr.reshape(-1)
    maskf = mask.astype(jnp.float32)

    pos128 = jnp.concatenate([pos, jnp.zeros((N, 125), jnp.float32)], axis=1)
    psrc = _sc_gather(pos128, nbr_flat)[:, :3].reshape(N, MAXNB, 3)
    delta = pos[:, None, :] - psrc
    d2e = jnp.sum(delta * delta, axis=-1)
    s = jnp.where(mask, d2e, 1.0)
    ew = jnp.sqrt(s)
    c = 0.5 * (jnp.cos(ew / CUTOFF * jnp.pi) + 1.0)
    cm = maskf * c

    h = params['emb'][z]

    wk = []
    for i in range(NI):
        w0 = params[f'b{i}_mlp0_w']
        w2 = params[f'b{i}_mlp2_w']
        wk.append(dict(
            w0t=_pad_rows(w0.T, 128),
            b0=params[f'b{i}_mlp0_b'][None, :],
            w2t=w2.T,
            b2=params[f'b{i}_mlp2_b'][None, :],
            w0=_pad_cols(w0, 128),
            w2=w2,
        ))

    saved = []
    for i in range(NI):
        k = wk[i]
        h1 = h @ params[f'b{i}_conv_lin1_w'].T
        g = _sc_gather(h1, nbr_flat)
        agg = _edge_fwd(ew, cm, g, k['w0t'], k['b0'], k['w2t'], k['b2'])
        u = h1 + agg
        v = u @ params[f'b{i}_conv_lin2_w'].T + params[f'b{i}_conv_lin2_b']
        sigV = jax.nn.sigmoid(v)
        w_ = _ssp(v)
        h = w_ @ params[f'b{i}_lin_w'].T + params[f'b{i}_lin_b']
        saved.append((g, sigV))

    y1 = h @ params['lin1_w'].T + params['lin1_b']
    sigY = jax.nn.sigmoid(y1)
    y2 = _ssp(y1)
    y3 = y2 @ params['lin2_w'].T + params['lin2_b']
    energy = jnp.sum(y3)

    zeros_nf = jnp.zeros((N, 128), jnp.float32)
    dy2 = jnp.broadcast_to(params['lin2_w'][0], y2.shape)
    dy1 = dy2 * sigY
    dh = dy1 @ params['lin1_w']

    dew_tot = jnp.zeros((N, MAXNB), jnp.float32)
    for i in range(NI - 1, -1, -1):
        g, sigV = saved[i]
        k = wk[i]
        dw_ = dh @ params[f'b{i}_lin_w']
        dv = dw_ * sigV
        du = dv @ params[f'b{i}_conv_lin2_w']
        dg, dew = _edge_bwd(ew, cm, maskf, g, du,
                            k['w0t'], k['b0'], k['w2t'], k['b2'],
                            k['w0'], k['w2'])
        dew_tot = dew_tot + dew
        sc = _sc_scatter(nbr_flat, dg, zeros_nf)
        dh1 = du + sc[:N] + sc[N:]
        dh = dh1 @ params[f'b{i}_conv_lin1_w']

    dd2e = maskf * dew_tot * (0.5 / ew)
    ddelta = 2.0 * dd2e[..., None] * delta
    dd128 = jnp.concatenate([ddelta.reshape(N * MAXNB, 3),
                             jnp.zeros((N * MAXNB, 125), jnp.float32)], axis=1)
    sc3 = _sc_scatter(nbr_flat, dd128, zeros_nf)
    dpos = ddelta.sum(axis=1) - (sc3[:N, :3] + sc3[N:, :3])

    return (energy, dpos)
